# baseline (device time: 70874 ns/iter reference)
import jax
import jax.numpy as jnp
from jax import lax
from jax.experimental import pallas as pl
from jax.experimental.pallas import tpu as pltpu

N_DEV = 8


def kernel(x, w_mat):
    m_per, k = x.shape
    _, n_per = w_mat.shape
    half = m_per // 2
    TOP = pl.ds(0, half)
    BOT = pl.ds(half, half)

    def body(x_ref, w_ref, out_ref, fwd_ref, bwd_ref, ant_ref, chd_ref,
             ftop_ref, fbot_ref, fsend, frecv, bsend, brecv, csend, crecv):
        my = lax.axis_index("i")

        def id_at(pos):
            pos = pos % N_DEV
            return jnp.where(pos < 4, pos, 11 - pos)

        p = jnp.where(my < 4, my, 11 - my)
        right = id_at(p + 1)
        left = id_at(p - 1)
        is_even = (p % 2) == 0
        is_odd = jnp.logical_not(is_even)
        partner = id_at(jnp.where(is_even, p + 3, p - 3))

        barrier_sem = pltpu.get_barrier_semaphore()
        for nbr in (left, right, partner):
            pl.semaphore_signal(
                barrier_sem, inc=1,
                device_id=(nbr,), device_id_type=pl.DeviceIdType.MESH,
            )
        pl.semaphore_wait(barrier_sem, 3)

        def gemm_silu_store(chunk, origin, row0=0):
            y = jnp.dot(chunk, w_ref[...], preferred_element_type=jnp.float32)
            out_ref[pl.ds(origin * m_per + row0, chunk.shape[0]), :] = (
                y * jax.nn.sigmoid(y)
            )

        def rdma(src, dst, ssem, rsem, dev):
            return pltpu.make_async_remote_copy(
                src_ref=src, dst_ref=dst, send_sem=ssem, recv_sem=rsem,
                device_id=(dev,), device_id_type=pl.DeviceIdType.MESH,
            )

        f0a = rdma(fwd_ref.at[0, TOP], fwd_ref.at[1, TOP],
                   fsend.at[0], frecv.at[0], right)
        f0b = rdma(fwd_ref.at[0, BOT], fwd_ref.at[1, BOT],
                   fsend.at[1], frecv.at[1], right)
        f1a = rdma(fwd_ref.at[1, TOP], fwd_ref.at[2, TOP],
                   fsend.at[2], frecv.at[2], right)
        f1b = rdma(fwd_ref.at[1, BOT], fwd_ref.at[2, BOT],
                   fsend.at[3], frecv.at[3], right)
        f2_odd = rdma(fwd_ref.at[2, TOP], ftop_ref,
                      fsend.at[4], frecv.at[4], right)
        f2_even = rdma(ant_ref.at[TOP], ftop_ref,
                       fsend.at[4], frecv.at[4], right)
        b0a = rdma(bwd_ref.at[0, TOP], bwd_ref.at[1, TOP],
                   bsend.at[0], brecv.at[0], left)
        b0b = rdma(bwd_ref.at[0, BOT], bwd_ref.at[1, BOT],
                   bsend.at[1], brecv.at[1], left)
        b1a = rdma(bwd_ref.at[1, TOP], bwd_ref.at[2, TOP],
                   bsend.at[2], brecv.at[2], left)
        b1b = rdma(bwd_ref.at[1, BOT], bwd_ref.at[2, BOT],
                   bsend.at[3], brecv.at[3], left)
        b2_odd = rdma(ant_ref.at[BOT], fbot_ref,
                      bsend.at[4], brecv.at[4], left)
        b2_even = rdma(bwd_ref.at[2, BOT], fbot_ref,
                       bsend.at[4], brecv.at[4], left)
        c_own = rdma(fwd_ref.at[0], chd_ref, csend.at[0], crecv.at[0], partner)
        c_ra_even = rdma(fwd_ref.at[1, TOP], ant_ref.at[TOP],
                         csend.at[1], crecv.at[1], partner)
        c_ra_odd = rdma(bwd_ref.at[1, TOP], ant_ref.at[TOP],
                        csend.at[1], crecv.at[1], partner)
        c_rb_even = rdma(fwd_ref.at[1, BOT], ant_ref.at[BOT],
                         csend.at[2], crecv.at[2], partner)
        c_rb_odd = rdma(bwd_ref.at[1, BOT], ant_ref.at[BOT],
                        csend.at[2], crecv.at[2], partner)

        fwd_ref[0] = x_ref[...]
        bwd_ref[0] = x_ref[...]
        f0a.start()
        f0b.start()
        b0a.start()
        b0b.start()
        c_own.start()

        gemm_silu_store(x_ref[...], my)

        f0a.wait_recv()
        f1a.start()
        b0a.wait_recv()
        b1a.start()

        @pl.when(is_odd)
        def _():
            c_ra_odd.start()

        f0b.wait_recv()
        f1b.start()
        b0b.wait_recv()
        b1b.start()

        @pl.when(is_odd)
        def _():
            c_rb_odd.start()

        @pl.when(is_even)
        def _():
            c_rb_even.start()
            c_ra_even.start()

        gemm_silu_store(fwd_ref[1], id_at(p - 1))
        gemm_silu_store(bwd_ref[1], id_at(p + 1))
        c_own.wait_recv()
        gemm_silu_store(chd_ref[...], id_at(jnp.where(is_even, p + 3, p - 3)))

        f1a.wait_recv()
        f1b.wait_recv()
        b1a.wait_recv()
        b1b.wait_recv()
        c_ra_odd.wait_recv()
        c_rb_odd.wait_recv()

        @pl.when(is_even)
        def _():
            f2_even.start()
            b2_even.start()

        @pl.when(is_odd)
        def _():
            f2_odd.start()
            b2_odd.start()

        gemm_silu_store(fwd_ref[2], id_at(p - 2))
        gemm_silu_store(bwd_ref[2], id_at(p + 2))
        gemm_silu_store(ant_ref[...], id_at(p + 4))

        far = id_at(jnp.where(is_even, p - 3, p + 3))
        f2_odd.wait_recv()
        gemm_silu_store(ftop_ref[...], far, row0=0)
        b2_odd.wait_recv()
        gemm_silu_store(fbot_ref[...], far, row0=half)

        for d in (f0a, f0b, f1a, f1b, f2_odd, b0a, b0b, b1a, b1b, b2_odd,
                  c_own, c_ra_odd, c_rb_odd):
            d.wait_send()

    out_shape = jax.ShapeDtypeStruct((N_DEV * m_per, n_per), jnp.float32)
    return pl.pallas_call(
        body,
        out_shape=out_shape,
        in_specs=[
            pl.BlockSpec(memory_space=pltpu.VMEM),
            pl.BlockSpec(memory_space=pltpu.VMEM),
        ],
        out_specs=pl.BlockSpec(memory_space=pltpu.VMEM),
        scratch_shapes=[
            pltpu.VMEM((3, m_per, k), jnp.float32),
            pltpu.VMEM((3, m_per, k), jnp.float32),
            pltpu.VMEM((m_per, k), jnp.float32),
            pltpu.VMEM((m_per, k), jnp.float32),
            pltpu.VMEM((half, k), jnp.float32),
            pltpu.VMEM((half, k), jnp.float32),
            pltpu.SemaphoreType.DMA((5,)),
            pltpu.SemaphoreType.DMA((5,)),
            pltpu.SemaphoreType.DMA((5,)),
            pltpu.SemaphoreType.DMA((5,)),
            pltpu.SemaphoreType.DMA((3,)),
            pltpu.SemaphoreType.DMA((3,)),
        ],
        compiler_params=pltpu.CompilerParams(collective_id=0),
    )(x, w_mat)


# device time: 67249 ns/iter; 1.0539x vs baseline; 1.0539x over previous
import jax
import jax.numpy as jnp
from jax import lax
from jax.experimental import pallas as pl
from jax.experimental.pallas import tpu as pltpu

N_DEV = 8


def kernel(x, w_mat):
    m_per, k = x.shape
    _, n_per = w_mat.shape
    half = m_per // 2
    TOP = pl.ds(0, half)
    BOT = pl.ds(half, half)

    def body(x_ref, w_ref, out_ref, fwd_ref, bwd_ref, ant_ref, chd_ref,
             ftop_ref, fbot_ref, fsend, frecv, bsend, brecv, csend, crecv):
        my = lax.axis_index("i")

        def id_at(pos):
            pos = pos % N_DEV
            return jnp.where(pos < 4, pos, 11 - pos)

        p = jnp.where(my < 4, my, 11 - my)
        right = id_at(p + 1)
        left = id_at(p - 1)
        is_even = (p % 2) == 0
        is_odd = jnp.logical_not(is_even)
        partner = id_at(jnp.where(is_even, p + 3, p - 3))

        barrier_sem = pltpu.get_barrier_semaphore()
        for nbr in (left, right, partner):
            pl.semaphore_signal(
                barrier_sem, inc=1,
                device_id=(nbr,), device_id_type=pl.DeviceIdType.MESH,
            )
        pl.semaphore_wait(barrier_sem, 3)

        def gemm_silu_store(chunk, origin, row0=0):
            y = jnp.dot(chunk, w_ref[...], preferred_element_type=jnp.float32)
            out_ref[pl.ds(origin * m_per + row0, chunk.shape[0]), :] = (
                y * jax.nn.sigmoid(y)
            )

        def rdma(src, dst, ssem, rsem, dev):
            return pltpu.make_async_remote_copy(
                src_ref=src, dst_ref=dst, send_sem=ssem, recv_sem=rsem,
                device_id=(dev,), device_id_type=pl.DeviceIdType.MESH,
            )

        f0a = rdma(x_ref.at[TOP], fwd_ref.at[0, TOP],
                   fsend.at[0], frecv.at[0], right)
        f0b = rdma(x_ref.at[BOT], fwd_ref.at[0, BOT],
                   fsend.at[1], frecv.at[1], right)
        f1a = rdma(fwd_ref.at[0, TOP], fwd_ref.at[1, TOP],
                   fsend.at[2], frecv.at[2], right)
        f1b = rdma(fwd_ref.at[0, BOT], fwd_ref.at[1, BOT],
                   fsend.at[3], frecv.at[3], right)
        f2_odd = rdma(fwd_ref.at[1, TOP], ftop_ref,
                      fsend.at[4], frecv.at[4], right)
        f2_even = rdma(ant_ref.at[TOP], ftop_ref,
                       fsend.at[4], frecv.at[4], right)
        b0a = rdma(x_ref.at[TOP], bwd_ref.at[0, TOP],
                   bsend.at[0], brecv.at[0], left)
        b0b = rdma(x_ref.at[BOT], bwd_ref.at[0, BOT],
                   bsend.at[1], brecv.at[1], left)
        b1a = rdma(bwd_ref.at[0, TOP], bwd_ref.at[1, TOP],
                   bsend.at[2], brecv.at[2], left)
        b1b = rdma(bwd_ref.at[0, BOT], bwd_ref.at[1, BOT],
                   bsend.at[3], brecv.at[3], left)
        b2_odd = rdma(ant_ref.at[BOT], fbot_ref,
                      bsend.at[4], brecv.at[4], left)
        b2_even = rdma(bwd_ref.at[1, BOT], fbot_ref,
                       bsend.at[4], brecv.at[4], left)
        c_own = rdma(x_ref, chd_ref, csend.at[0], crecv.at[0], partner)
        c_ra_even = rdma(fwd_ref.at[0, TOP], ant_ref.at[TOP],
                         csend.at[1], crecv.at[1], partner)
        c_ra_odd = rdma(bwd_ref.at[0, TOP], ant_ref.at[TOP],
                        csend.at[1], crecv.at[1], partner)
        c_rb_even = rdma(fwd_ref.at[0, BOT], ant_ref.at[BOT],
                         csend.at[2], crecv.at[2], partner)
        c_rb_odd = rdma(bwd_ref.at[0, BOT], ant_ref.at[BOT],
                        csend.at[2], crecv.at[2], partner)

        f0a.start()
        f0b.start()
        b0a.start()
        b0b.start()
        c_own.start()

        gemm_silu_store(x_ref[...], my)

        f0a.wait_recv()
        f1a.start()
        b0a.wait_recv()
        b1a.start()

        @pl.when(is_odd)
        def _():
            c_ra_odd.start()

        f0b.wait_recv()
        f1b.start()
        b0b.wait_recv()
        b1b.start()

        @pl.when(is_odd)
        def _():
            c_rb_odd.start()

        @pl.when(is_even)
        def _():
            c_rb_even.start()
            c_ra_even.start()

        gemm_silu_store(fwd_ref[0], id_at(p - 1))
        gemm_silu_store(bwd_ref[0], id_at(p + 1))
        c_own.wait_recv()
        gemm_silu_store(chd_ref[...], id_at(jnp.where(is_even, p + 3, p - 3)))

        @pl.when(is_even)
        def _():
            c_ra_even.wait_recv()
            f2_even.start()

        @pl.when(is_odd)
        def _():
            c_rb_odd.wait_recv()
            b2_odd.start()

        f1a.wait_recv()

        @pl.when(is_odd)
        def _():
            f2_odd.start()

        b1b.wait_recv()

        @pl.when(is_even)
        def _():
            b2_even.start()

        f1b.wait_recv()
        b1a.wait_recv()

        @pl.when(is_even)
        def _():
            c_rb_even.wait_recv()

        @pl.when(is_odd)
        def _():
            c_ra_odd.wait_recv()

        gemm_silu_store(fwd_ref[1], id_at(p - 2))
        gemm_silu_store(bwd_ref[1], id_at(p + 2))
        gemm_silu_store(ant_ref[...], id_at(p + 4))

        far = id_at(jnp.where(is_even, p - 3, p + 3))
        f2_odd.wait_recv()
        gemm_silu_store(ftop_ref[...], far, row0=0)
        b2_odd.wait_recv()
        gemm_silu_store(fbot_ref[...], far, row0=half)

        for d in (f0a, f0b, f1a, f1b, f2_odd, b0a, b0b, b1a, b1b, b2_odd,
                  c_own, c_ra_odd, c_rb_odd):
            d.wait_send()

    out_shape = jax.ShapeDtypeStruct((N_DEV * m_per, n_per), jnp.float32)
    return pl.pallas_call(
        body,
        out_shape=out_shape,
        in_specs=[
            pl.BlockSpec(memory_space=pltpu.VMEM),
            pl.BlockSpec(memory_space=pltpu.VMEM),
        ],
        out_specs=pl.BlockSpec(memory_space=pltpu.VMEM),
        scratch_shapes=[
            pltpu.VMEM((2, m_per, k), jnp.float32),
            pltpu.VMEM((2, m_per, k), jnp.float32),
            pltpu.VMEM((m_per, k), jnp.float32),
            pltpu.VMEM((m_per, k), jnp.float32),
            pltpu.VMEM((half, k), jnp.float32),
            pltpu.VMEM((half, k), jnp.float32),
            pltpu.SemaphoreType.DMA((5,)),
            pltpu.SemaphoreType.DMA((5,)),
            pltpu.SemaphoreType.DMA((5,)),
            pltpu.SemaphoreType.DMA((5,)),
            pltpu.SemaphoreType.DMA((3,)),
            pltpu.SemaphoreType.DMA((3,)),
        ],
        compiler_params=pltpu.CompilerParams(collective_id=0),
    )(x, w_mat)


# device time: 64941 ns/iter; 1.0914x vs baseline; 1.0355x over previous
import jax
import jax.numpy as jnp
from jax import lax
from jax.experimental import pallas as pl
from jax.experimental.pallas import tpu as pltpu

N_DEV = 8
R1 = 88
R2 = 80
R3 = 88


def kernel(x, w_mat):
    m_per, k = x.shape
    _, n_per = w_mat.shape
    half = m_per // 2
    TOP = pl.ds(0, half)
    BOT = pl.ds(half, half)
    FAR1 = pl.ds(0, R1)
    FAR2 = pl.ds(R1, R2)
    FAR3 = pl.ds(R1 + R2, R3)

    def body(x_ref, w_ref, out_ref, fwd_ref, bwd_ref, ant_ref, chd_ref,
             f1_ref, f2_ref, f3_ref, fsend, frecv, bsend, brecv, csend, crecv):
        my = lax.axis_index("i")

        def id_at(pos):
            pos = pos % N_DEV
            return jnp.where(pos < 4, pos, 11 - pos)

        p = jnp.where(my < 4, my, 11 - my)
        right = id_at(p + 1)
        left = id_at(p - 1)
        is_even = (p % 2) == 0
        is_odd = jnp.logical_not(is_even)
        partner = id_at(jnp.where(is_even, p + 3, p - 3))

        barrier_sem = pltpu.get_barrier_semaphore()
        for nbr in (left, right, partner):
            pl.semaphore_signal(
                barrier_sem, inc=1,
                device_id=(nbr,), device_id_type=pl.DeviceIdType.MESH,
            )
        pl.semaphore_wait(barrier_sem, 3)

        def gemm_silu_store(chunk, origin, row0=0):
            y = jnp.dot(chunk, w_ref[...], preferred_element_type=jnp.float32)
            out_ref[pl.ds(origin * m_per + row0, chunk.shape[0]), :] = (
                y * jax.nn.sigmoid(y)
            )

        def rdma(src, dst, ssem, rsem, dev):
            return pltpu.make_async_remote_copy(
                src_ref=src, dst_ref=dst, send_sem=ssem, recv_sem=rsem,
                device_id=(dev,), device_id_type=pl.DeviceIdType.MESH,
            )

        f0a = rdma(x_ref.at[TOP], fwd_ref.at[0, TOP],
                   fsend.at[0], frecv.at[0], right)
        f0b = rdma(x_ref.at[BOT], fwd_ref.at[0, BOT],
                   fsend.at[1], frecv.at[1], right)
        f1a = rdma(fwd_ref.at[0, TOP], fwd_ref.at[1, TOP],
                   fsend.at[2], frecv.at[2], right)
        f1b = rdma(fwd_ref.at[0, BOT], fwd_ref.at[1, BOT],
                   fsend.at[3], frecv.at[3], right)
        ff_odd = rdma(fwd_ref.at[1, FAR1], f1_ref,
                      fsend.at[4], frecv.at[4], right)
        ff_even = rdma(ant_ref.at[FAR1], f1_ref,
                       fsend.at[4], frecv.at[4], right)
        b0a = rdma(x_ref.at[TOP], bwd_ref.at[0, TOP],
                   bsend.at[0], brecv.at[0], left)
        b0b = rdma(x_ref.at[BOT], bwd_ref.at[0, BOT],
                   bsend.at[1], brecv.at[1], left)
        b1a = rdma(bwd_ref.at[0, TOP], bwd_ref.at[1, TOP],
                   bsend.at[2], brecv.at[2], left)
        b1b = rdma(bwd_ref.at[0, BOT], bwd_ref.at[1, BOT],
                   bsend.at[3], brecv.at[3], left)
        bf_odd = rdma(ant_ref.at[FAR3], f3_ref,
                      bsend.at[4], brecv.at[4], left)
        bf_even = rdma(bwd_ref.at[1, FAR3], f3_ref,
                       bsend.at[4], brecv.at[4], left)
        c_own = rdma(x_ref, chd_ref, csend.at[0], crecv.at[0], partner)
        c_ra_even = rdma(fwd_ref.at[0, TOP], ant_ref.at[TOP],
                         csend.at[1], crecv.at[1], partner)
        c_ra_odd = rdma(bwd_ref.at[0, TOP], ant_ref.at[TOP],
                        csend.at[1], crecv.at[1], partner)
        c_rb_even = rdma(fwd_ref.at[0, BOT], ant_ref.at[BOT],
                         csend.at[2], crecv.at[2], partner)
        c_rb_odd = rdma(bwd_ref.at[0, BOT], ant_ref.at[BOT],
                        csend.at[2], crecv.at[2], partner)
        cf_even = rdma(fwd_ref.at[1, FAR2], f2_ref,
                       csend.at[3], crecv.at[3], partner)
        cf_odd = rdma(bwd_ref.at[1, FAR2], f2_ref,
                      csend.at[3], crecv.at[3], partner)

        f0a.start()
        f0b.start()
        b0a.start()
        b0b.start()
        c_own.start()

        gemm_silu_store(x_ref[...], my)

        f0a.wait_recv()
        f1a.start()
        b0a.wait_recv()
        b1a.start()

        @pl.when(is_odd)
        def _():
            c_ra_odd.start()

        f0b.wait_recv()
        f1b.start()
        b0b.wait_recv()
        b1b.start()

        @pl.when(is_odd)
        def _():
            c_rb_odd.start()

        @pl.when(is_even)
        def _():
            c_rb_even.start()
            c_ra_even.start()

        gemm_silu_store(fwd_ref[0], id_at(p - 1))
        gemm_silu_store(bwd_ref[0], id_at(p + 1))
        c_own.wait_recv()
        gemm_silu_store(chd_ref[...], id_at(jnp.where(is_even, p + 3, p - 3)))

        @pl.when(is_even)
        def _():
            c_ra_even.wait_recv()
            ff_even.start()

        @pl.when(is_odd)
        def _():
            c_rb_odd.wait_recv()
            bf_odd.start()

        f1a.wait_recv()

        @pl.when(is_odd)
        def _():
            ff_odd.start()

        b1a.wait_recv()
        f1b.wait_recv()
        b1b.wait_recv()

        @pl.when(is_even)
        def _():
            bf_even.start()
            cf_even.start()

        @pl.when(is_odd)
        def _():
            cf_odd.start()

        @pl.when(is_even)
        def _():
            c_rb_even.wait_recv()

        @pl.when(is_odd)
        def _():
            c_ra_odd.wait_recv()

        gemm_silu_store(fwd_ref[1], id_at(p - 2))
        gemm_silu_store(bwd_ref[1], id_at(p + 2))
        gemm_silu_store(ant_ref[...], id_at(p + 4))

        far = id_at(jnp.where(is_even, p - 3, p + 3))
        ff_odd.wait_recv()
        gemm_silu_store(f1_ref[...], far, row0=0)
        cf_odd.wait_recv()
        gemm_silu_store(f2_ref[...], far, row0=R1)
        bf_odd.wait_recv()
        gemm_silu_store(f3_ref[...], far, row0=R1 + R2)

        for d in (f0a, f0b, f1a, f1b, ff_odd, b0a, b0b, b1a, b1b, bf_odd,
                  c_own, c_ra_odd, c_rb_odd, cf_odd):
            d.wait_send()

    out_shape = jax.ShapeDtypeStruct((N_DEV * m_per, n_per), jnp.float32)
    return pl.pallas_call(
        body,
        out_shape=out_shape,
        in_specs=[
            pl.BlockSpec(memory_space=pltpu.VMEM),
            pl.BlockSpec(memory_space=pltpu.VMEM),
        ],
        out_specs=pl.BlockSpec(memory_space=pltpu.VMEM),
        scratch_shapes=[
            pltpu.VMEM((2, m_per, k), jnp.float32),
            pltpu.VMEM((2, m_per, k), jnp.float32),
            pltpu.VMEM((m_per, k), jnp.float32),
            pltpu.VMEM((m_per, k), jnp.float32),
            pltpu.VMEM((R1, k), jnp.float32),
            pltpu.VMEM((R2, k), jnp.float32),
            pltpu.VMEM((R3, k), jnp.float32),
            pltpu.SemaphoreType.DMA((5,)),
            pltpu.SemaphoreType.DMA((5,)),
            pltpu.SemaphoreType.DMA((5,)),
            pltpu.SemaphoreType.DMA((5,)),
            pltpu.SemaphoreType.DMA((4,)),
            pltpu.SemaphoreType.DMA((4,)),
        ],
        compiler_params=pltpu.CompilerParams(collective_id=0),
    )(x, w_mat)


# device time: 64928 ns/iter; 1.0916x vs baseline; 1.0002x over previous
import jax
import jax.numpy as jnp
from jax import lax
from jax.experimental import pallas as pl
from jax.experimental.pallas import tpu as pltpu

N_DEV = 8
R1 = 88
R2 = 80
R3 = 88


def kernel(x, w_mat):
    m_per, k = x.shape
    _, n_per = w_mat.shape
    half = m_per // 2
    TOP = pl.ds(0, half)
    BOT = pl.ds(half, half)
    FAR1 = pl.ds(0, R1)
    FAR2 = pl.ds(R1, R2)
    FAR3 = pl.ds(R1 + R2, R3)

    def body(x_ref, w_ref, out_ref, fwd_ref, bwd_ref, ant_ref, chd_ref,
             f1_ref, f2_ref, f3_ref, fsend, frecv, bsend, brecv, csend, crecv):
        my = lax.axis_index("i")

        def id_at(pos):
            pos = pos % N_DEV
            return jnp.where(pos < 4, pos, 11 - pos)

        p = jnp.where(my < 4, my, 11 - my)
        right = id_at(p + 1)
        left = id_at(p - 1)
        is_even = (p % 2) == 0
        is_odd = jnp.logical_not(is_even)
        partner = id_at(jnp.where(is_even, p + 3, p - 3))

        barrier_sem = pltpu.get_barrier_semaphore()
        for nbr in (left, right, partner):
            pl.semaphore_signal(
                barrier_sem, inc=1,
                device_id=(nbr,), device_id_type=pl.DeviceIdType.MESH,
            )
        pl.semaphore_wait(barrier_sem, 3)

        def gemm_silu_store(chunk, origin, row0=0):
            y = jnp.dot(chunk, w_ref[...], preferred_element_type=jnp.float32)
            out_ref[pl.ds(origin * m_per + row0, chunk.shape[0]), :] = (
                y * jax.nn.sigmoid(y)
            )

        def rdma(src, dst, ssem, rsem, dev):
            return pltpu.make_async_remote_copy(
                src_ref=src, dst_ref=dst, send_sem=ssem, recv_sem=rsem,
                device_id=(dev,), device_id_type=pl.DeviceIdType.MESH,
            )

        f0a = rdma(x_ref.at[TOP], fwd_ref.at[0, TOP],
                   fsend.at[0], frecv.at[0], right)
        f0b = rdma(x_ref.at[BOT], fwd_ref.at[0, BOT],
                   fsend.at[1], frecv.at[1], right)
        f1a = rdma(fwd_ref.at[0, TOP], fwd_ref.at[1, TOP],
                   fsend.at[2], frecv.at[2], right)
        f1b = rdma(fwd_ref.at[0, BOT], fwd_ref.at[1, BOT],
                   fsend.at[3], frecv.at[3], right)
        ff_odd = rdma(fwd_ref.at[1, FAR1], f1_ref,
                      fsend.at[4], frecv.at[4], right)
        ff_even = rdma(ant_ref.at[FAR1], f1_ref,
                       fsend.at[4], frecv.at[4], right)
        b0a = rdma(x_ref.at[TOP], bwd_ref.at[0, TOP],
                   bsend.at[0], brecv.at[0], left)
        b0b = rdma(x_ref.at[BOT], bwd_ref.at[0, BOT],
                   bsend.at[1], brecv.at[1], left)
        b1a = rdma(bwd_ref.at[0, TOP], bwd_ref.at[1, TOP],
                   bsend.at[2], brecv.at[2], left)
        b1b = rdma(bwd_ref.at[0, BOT], bwd_ref.at[1, BOT],
                   bsend.at[3], brecv.at[3], left)
        bf_odd = rdma(ant_ref.at[FAR3], f3_ref,
                      bsend.at[4], brecv.at[4], left)
        bf_even = rdma(bwd_ref.at[1, FAR3], f3_ref,
                       bsend.at[4], brecv.at[4], left)
        c_own = rdma(x_ref, chd_ref, csend.at[0], crecv.at[0], partner)
        c_ra_even = rdma(fwd_ref.at[0, TOP], ant_ref.at[TOP],
                         csend.at[1], crecv.at[1], partner)
        c_ra_odd = rdma(bwd_ref.at[0, TOP], ant_ref.at[TOP],
                        csend.at[1], crecv.at[1], partner)
        c_rb_even = rdma(fwd_ref.at[0, BOT], ant_ref.at[BOT],
                         csend.at[2], crecv.at[2], partner)
        c_rb_odd = rdma(bwd_ref.at[0, BOT], ant_ref.at[BOT],
                        csend.at[2], crecv.at[2], partner)
        cf_even = rdma(fwd_ref.at[1, FAR2], f2_ref,
                       csend.at[3], crecv.at[3], partner)
        cf_odd = rdma(bwd_ref.at[1, FAR2], f2_ref,
                      csend.at[3], crecv.at[3], partner)

        f0a.start()
        f0b.start()
        b0a.start()
        b0b.start()
        c_own.start()

        gemm_silu_store(x_ref[...], my)

        f0a.wait_recv()
        f1a.start()
        b0a.wait_recv()
        b1a.start()

        @pl.when(is_odd)
        def _():
            c_ra_odd.start()

        f0b.wait_recv()
        f1b.start()
        b0b.wait_recv()
        b1b.start()

        @pl.when(is_odd)
        def _():
            c_rb_odd.start()

        @pl.when(is_even)
        def _():
            c_rb_even.start()
            c_ra_even.start()

        gemm_silu_store(fwd_ref[0], id_at(p - 1))
        gemm_silu_store(bwd_ref[0], id_at(p + 1))
        c_own.wait_recv()
        gemm_silu_store(chd_ref[...], id_at(jnp.where(is_even, p + 3, p - 3)))

        f1a.wait_recv()

        @pl.when(is_odd)
        def _():
            ff_odd.start()

        b1a.wait_recv()
        gemm_silu_store(fwd_ref[1, TOP], id_at(p - 2), row0=0)
        gemm_silu_store(bwd_ref[1, TOP], id_at(p + 2), row0=0)

        @pl.when(is_even)
        def _():
            c_ra_even.wait_recv()
            ff_even.start()
            gemm_silu_store(ant_ref[TOP], id_at(p + 4), row0=0)

        @pl.when(is_odd)
        def _():
            c_rb_odd.wait_recv()
            bf_odd.start()
            gemm_silu_store(ant_ref[BOT], id_at(p + 4), row0=half)

        f1b.wait_recv()
        b1b.wait_recv()

        @pl.when(is_even)
        def _():
            bf_even.start()
            cf_even.start()

        @pl.when(is_odd)
        def _():
            cf_odd.start()

        gemm_silu_store(fwd_ref[1, BOT], id_at(p - 2), row0=half)
        gemm_silu_store(bwd_ref[1, BOT], id_at(p + 2), row0=half)

        @pl.when(is_even)
        def _():
            c_rb_even.wait_recv()
            gemm_silu_store(ant_ref[BOT], id_at(p + 4), row0=half)

        @pl.when(is_odd)
        def _():
            c_ra_odd.wait_recv()
            gemm_silu_store(ant_ref[TOP], id_at(p + 4), row0=0)

        far = id_at(jnp.where(is_even, p - 3, p + 3))
        ff_odd.wait_recv()
        gemm_silu_store(f1_ref[...], far, row0=0)
        cf_odd.wait_recv()
        gemm_silu_store(f2_ref[...], far, row0=R1)
        bf_odd.wait_recv()
        gemm_silu_store(f3_ref[...], far, row0=R1 + R2)

        for d in (f0a, f0b, f1a, f1b, ff_odd, b0a, b0b, b1a, b1b, bf_odd,
                  c_own, c_ra_odd, c_rb_odd, cf_odd):
            d.wait_send()

    out_shape = jax.ShapeDtypeStruct((N_DEV * m_per, n_per), jnp.float32)
    return pl.pallas_call(
        body,
        out_shape=out_shape,
        in_specs=[
            pl.BlockSpec(memory_space=pltpu.VMEM),
            pl.BlockSpec(memory_space=pltpu.VMEM),
        ],
        out_specs=pl.BlockSpec(memory_space=pltpu.VMEM),
        scratch_shapes=[
            pltpu.VMEM((2, m_per, k), jnp.float32),
            pltpu.VMEM((2, m_per, k), jnp.float32),
            pltpu.VMEM((m_per, k), jnp.float32),
            pltpu.VMEM((m_per, k), jnp.float32),
            pltpu.VMEM((R1, k), jnp.float32),
            pltpu.VMEM((R2, k), jnp.float32),
            pltpu.VMEM((R3, k), jnp.float32),
            pltpu.SemaphoreType.DMA((5,)),
            pltpu.SemaphoreType.DMA((5,)),
            pltpu.SemaphoreType.DMA((5,)),
            pltpu.SemaphoreType.DMA((5,)),
            pltpu.SemaphoreType.DMA((4,)),
            pltpu.SemaphoreType.DMA((4,)),
        ],
        compiler_params=pltpu.CompilerParams(collective_id=0),
    )(x, w_mat)
